# Initial kernel scaffold; baseline (speedup 1.0000x reference)
import jax
import jax.numpy as jnp
from jax.experimental import pallas as pl


def kernel(x, edge_index, batch, t, Wl0, bl0, Wr0, Wl1, bl1, Wr1, Wl2, bl2, Wr2, Wt1, bt1, Wt2, bt2):
    raise NotImplementedError("write your pallas kernel here")


# speedup vs baseline: 19.7260x; 19.7260x over previous
import jax
import jax.numpy as jnp
from jax import lax
from jax.experimental import pallas as pl
from jax.experimental.pallas import tpu as pltpu
from jax.experimental.pallas import tpu_sc as plsc

N = 49152
E = 786432
NB = 12

_NROWS_TILE = N // 16


def _gather_scatter_pipeline(tbl, src, dst, acc, base, nchunks, ch,
                             s0, s1, d0, d1, r0, r1, gs0, gs1, ss0, ss1):
    pltpu.sync_copy(src.at[pl.ds(base, ch)], s0)
    pltpu.sync_copy(dst.at[pl.ds(base, ch)], d0)
    pltpu.async_copy(tbl.at[s0], r0, gs0)

    def body(j, carry):
        i0 = 2 * j
        @pl.when(j > 0)
        def _():
            pltpu.make_async_copy(r1, acc.at[d1], ss1).wait()
        pltpu.sync_copy(src.at[pl.ds(base + (i0 + 1) * ch, ch)], s1)
        pltpu.sync_copy(dst.at[pl.ds(base + (i0 + 1) * ch, ch)], d1)
        pltpu.async_copy(tbl.at[s1], r1, gs1)
        pltpu.make_async_copy(tbl.at[s0], r0, gs0).wait()
        pltpu.async_copy(r0, acc.at[d0], ss0, add=True)

        @pl.when(j < nchunks // 2 - 1)
        def _():
            pltpu.make_async_copy(r0, acc.at[d0], ss0).wait()
            pltpu.sync_copy(src.at[pl.ds(base + (i0 + 2) * ch, ch)], s0)
            pltpu.sync_copy(dst.at[pl.ds(base + (i0 + 2) * ch, ch)], d0)
            pltpu.async_copy(tbl.at[s0], r0, gs0)
        pltpu.make_async_copy(tbl.at[s1], r1, gs1).wait()
        pltpu.async_copy(r1, acc.at[d1], ss1, add=True)
        return carry

    lax.fori_loop(0, nchunks // 2, body, 0)
    pltpu.make_async_copy(r0, acc.at[d0], ss0).wait()
    pltpu.make_async_copy(r1, acc.at[d1], ss1).wait()


def _zero_acc(zeros, acc, s, bounce, wb):
    def z(j, carry):
        t0 = s * _NROWS_TILE + j * wb
        pltpu.sync_copy(zeros.at[pl.ds(t0, wb)], bounce)
        pltpu.sync_copy(bounce, acc.at[pl.ds(t0, wb)])
        return carry
    lax.fori_loop(0, _NROWS_TILE // wb, z, 0)


_E_T4 = E // 32
_CH4 = 2048
_WB4 = 1536


def _edge4_body(table, src, dst, zeros, out_a, out_b,
                s0, s1, d0, d1, r0, r1, acc, gs0, gs1, ss0, ss1):
    c = lax.axis_index("c")
    s = lax.axis_index("s")
    bounce = r0.at[pl.ds(0, _WB4)]
    _zero_acc(zeros, acc, s, bounce, _WB4)
    plsc.subcore_barrier()

    base = (c * 16 + s) * _E_T4
    _gather_scatter_pipeline(table, src, dst, acc, base,
                             _E_T4 // _CH4, _CH4,
                             s0, s1, d0, d1, r0, r1, gs0, gs1, ss0, ss1)
    plsc.subcore_barrier()

    def wb(j, carry):
        t0 = s * _NROWS_TILE + j * _WB4
        pltpu.sync_copy(acc.at[pl.ds(t0, _WB4)], bounce)

        @pl.when(c == 0)
        def _():
            pltpu.sync_copy(bounce, out_a.at[pl.ds(t0, _WB4)])

        @pl.when(c == 1)
        def _():
            pltpu.sync_copy(bounce, out_b.at[pl.ds(t0, _WB4)])
        return carry

    lax.fori_loop(0, _NROWS_TILE // _WB4, wb, 0)


_edge4 = pl.kernel(
    _edge4_body,
    out_type=[jax.ShapeDtypeStruct((N, 8), jnp.float32)] * 2,
    mesh=plsc.VectorSubcoreMesh(core_axis_name="c", subcore_axis_name="s",
                                num_cores=2, num_subcores=16),
    compiler_params=pltpu.CompilerParams(use_tc_tiling_on_sc=False),
    scratch_types=[
        pltpu.VMEM((_CH4,), jnp.int32),
        pltpu.VMEM((_CH4,), jnp.int32),
        pltpu.VMEM((_CH4,), jnp.int32),
        pltpu.VMEM((_CH4,), jnp.int32),
        pltpu.VMEM((_CH4, 8), jnp.float32),
        pltpu.VMEM((_CH4, 8), jnp.float32),
        pltpu.VMEM_SHARED((N, 8), jnp.float32),
        pltpu.SemaphoreType.DMA,
        pltpu.SemaphoreType.DMA,
        pltpu.SemaphoreType.DMA,
        pltpu.SemaphoreType.DMA,
    ],
)


_E_T32 = E // 16
_CH32 = 384
_WB32 = 384


def _edge32_body(h_a, h_b, src, dst, zeros, out_a, out_b,
                 s0, s1, d0, d1, r0, r1, acc, gs0, gs1, ss0, ss1):
    c = lax.axis_index("c")
    s = lax.axis_index("s")
    _zero_acc(zeros, acc, s, r0, _WB32)
    plsc.subcore_barrier()

    base = s * _E_T32
    nchunks = _E_T32 // _CH32

    @pl.when(c == 0)
    def _():
        _gather_scatter_pipeline(h_a, src, dst, acc, base, nchunks, _CH32,
                                 s0, s1, d0, d1, r0, r1, gs0, gs1, ss0, ss1)

    @pl.when(c == 1)
    def _():
        _gather_scatter_pipeline(h_b, src, dst, acc, base, nchunks, _CH32,
                                 s0, s1, d0, d1, r0, r1, gs0, gs1, ss0, ss1)

    plsc.subcore_barrier()

    def wb(j, carry):
        t0 = s * _NROWS_TILE + j * _WB32
        pltpu.sync_copy(acc.at[pl.ds(t0, _WB32)], r0)

        @pl.when(c == 0)
        def _():
            pltpu.sync_copy(r0, out_a.at[pl.ds(t0, _WB32)])

        @pl.when(c == 1)
        def _():
            pltpu.sync_copy(r0, out_b.at[pl.ds(t0, _WB32)])
        return carry

    lax.fori_loop(0, _NROWS_TILE // _WB32, wb, 0)


_edge32 = pl.kernel(
    _edge32_body,
    out_type=[jax.ShapeDtypeStruct((N, 32), jnp.float32)] * 2,
    mesh=plsc.VectorSubcoreMesh(core_axis_name="c", subcore_axis_name="s",
                                num_cores=2, num_subcores=16),
    compiler_params=pltpu.CompilerParams(use_tc_tiling_on_sc=False),
    scratch_types=[
        pltpu.VMEM((_CH32,), jnp.int32),
        pltpu.VMEM((_CH32,), jnp.int32),
        pltpu.VMEM((_CH32,), jnp.int32),
        pltpu.VMEM((_CH32,), jnp.int32),
        pltpu.VMEM((_CH32, 32), jnp.float32),
        pltpu.VMEM((_CH32, 32), jnp.float32),
        pltpu.VMEM_SHARED((N, 32), jnp.float32),
        pltpu.SemaphoreType.DMA,
        pltpu.SemaphoreType.DMA,
        pltpu.SemaphoreType.DMA,
        pltpu.SemaphoreType.DMA,
    ],
)


_BN = 2048


def _silu(x):
    return x * jax.nn.sigmoid(x)


def _tmod_body(t_ref, w1_ref, b1_ref, w2_ref, b2_ref, o_ref):
    h = t_ref[...] * w1_ref[...] + b1_ref[...]
    h = _silu(h)
    o_ref[...] = (
        jnp.dot(h, w2_ref[...], preferred_element_type=jnp.float32)
        + b2_ref[...]
    )


def _onehot(bt):
    return (bt == lax.broadcasted_iota(jnp.int32, (_BN, NB), 1)).astype(
        jnp.float32
    )


def _tc1_body(x_ref, pa_ref, pb_ref, bt_ref, g_ref, be_ref,
              wl_ref, wr_ref, bl_ref, ha_ref, hb_ref, invd_ref):
    p = pa_ref[...] + pb_ref[...]
    invd = 1.0 / jnp.maximum(p[:, 3:4], 1.0)
    agg = p[:, 0:3] * invd
    h = (jnp.dot(agg, wl_ref[...], preferred_element_type=jnp.float32)
         + jnp.dot(x_ref[...], wr_ref[...], preferred_element_type=jnp.float32)
         + bl_ref[...])
    oh = _onehot(bt_ref[...])
    gm = jnp.dot(oh, g_ref[...], preferred_element_type=jnp.float32)
    bb = jnp.dot(oh, be_ref[...], preferred_element_type=jnp.float32)
    h = _silu(h * (1.0 + gm) + bb)
    ha_ref[...] = h[:, :32]
    hb_ref[...] = h[:, 32:]
    invd_ref[...] = invd


def _tc2_body(aa_ref, ab_ref, ha_ref, hb_ref, invd_ref, bt_ref,
              g_ref, be_ref, wl_ref, wr_ref, bl_ref, wl2_ref, wr2_ref,
              y_ref, r_ref):
    agg = jnp.concatenate([aa_ref[...], ab_ref[...]], axis=1) * invd_ref[...]
    h1 = jnp.concatenate([ha_ref[...], hb_ref[...]], axis=1)
    h = (jnp.dot(agg, wl_ref[...], preferred_element_type=jnp.float32)
         + jnp.dot(h1, wr_ref[...], preferred_element_type=jnp.float32)
         + bl_ref[...])
    oh = _onehot(bt_ref[...])
    gm = jnp.dot(oh, g_ref[...], preferred_element_type=jnp.float32)
    bb = jnp.dot(oh, be_ref[...], preferred_element_type=jnp.float32)
    h = _silu(h * (1.0 + gm) + bb)
    y_ref[...] = jnp.dot(h, wl2_ref[...], preferred_element_type=jnp.float32)
    r_ref[...] = jnp.dot(h, wr2_ref[...], preferred_element_type=jnp.float32)


def _tc3_body(qa_ref, qb_ref, invd_ref, r_ref, bt_ref,
              g_ref, be_ref, bl_ref, o_ref):
    agg = (qa_ref[...] + qb_ref[...])[:, 0:3] * invd_ref[...]
    pre = agg + bl_ref[...] + r_ref[...][:, 0:3]
    oh = _onehot(bt_ref[...])
    gm = jnp.dot(oh, g_ref[...], preferred_element_type=jnp.float32)
    bb = jnp.dot(oh, be_ref[...], preferred_element_type=jnp.float32)
    o_ref[...] = pre * (1.0 + gm) + bb


def _col(shape):
    return pl.BlockSpec((_BN,) + shape[1:],
                        lambda i: (i,) + (0,) * (len(shape) - 1))


def _full(shape):
    return pl.BlockSpec(shape, lambda i: (0,) * len(shape))


def _tc_call(body, in_shapes, out_shapes):
    return pl.pallas_call(
        body,
        grid=(N // _BN,),
        in_specs=[_col(s) if s[0] == N else _full(s) for s in in_shapes],
        out_specs=[_col(s) for s in out_shapes],
        out_shape=[jax.ShapeDtypeStruct(s, jnp.float32) for s in out_shapes],
    )


_tmod_call = pl.pallas_call(
    _tmod_body,
    out_shape=jax.ShapeDtypeStruct((NB, 262), jnp.float32),
)

_tc1_call = _tc_call(
    _tc1_body,
    [(N, 3), (N, 8), (N, 8), (N, 1), (NB, 64), (NB, 64), (3, 64), (3, 64),
     (1, 64)],
    [(N, 32), (N, 32), (N, 1)],
)

_tc2_call = _tc_call(
    _tc2_body,
    [(N, 32), (N, 32), (N, 32), (N, 32), (N, 1), (N, 1), (NB, 64), (NB, 64),
     (64, 64), (64, 64), (1, 64), (64, 8), (64, 4)],
    [(N, 8), (N, 4)],
)

_tc3_call = _tc_call(
    _tc3_body,
    [(N, 8), (N, 8), (N, 1), (N, 4), (N, 1), (NB, 3), (NB, 3), (1, 3)],
    [(N, 3)],
)


def kernel(x, edge_index, batch, t, Wl0, bl0, Wr0, Wl1, bl1, Wr1,
           Wl2, bl2, Wr2, Wt1, bt1, Wt2, bt2):
    f32 = jnp.float32
    src1, dst1 = edge_index[0], edge_index[1]
    batch2d = batch.reshape(N, 1)
    x8 = jnp.concatenate([x, jnp.ones((N, 1), f32),
                          jnp.zeros((N, 4), f32)], axis=1)
    zeros8 = jnp.zeros((N, 8), f32)
    zeros32 = jnp.zeros((N, 32), f32)
    t2 = t.reshape(NB, 1)

    wl0t, wr0t = Wl0.T, Wr0.T
    wl1t, wr1t = Wl1.T, Wr1.T
    wl2t8 = jnp.pad(Wl2.T, ((0, 0), (0, 5)))
    wr2t4 = jnp.pad(Wr2.T, ((0, 0), (0, 1)))
    bl0r, bl1r, bl2r = bl0.reshape(1, 64), bl1.reshape(1, 64), bl2.reshape(1, 3)
    wt1t, wt2t = Wt1.T, Wt2.T
    bt1r, bt2r = bt1.reshape(1, 128), bt2.reshape(1, 262)

    tmod = _tmod_call(t2, wt1t, bt1r, wt2t, bt2r)
    g0, b0 = tmod[:, 0:64], tmod[:, 64:128]
    g1, b1 = tmod[:, 128:192], tmod[:, 192:256]
    g2, b2 = tmod[:, 256:259], tmod[:, 259:262]

    pa, pb = _edge4(x8, src1, dst1, zeros8)
    ha, hb, invd = _tc1_call(x, pa, pb, batch2d, g0, b0, wl0t, wr0t, bl0r)
    aa, ab = _edge32(ha, hb, src1, dst1, zeros32)
    y2, r2 = _tc2_call(aa, ab, ha, hb, invd, batch2d, g1, b1,
                       wl1t, wr1t, bl1r, wl2t8, wr2t4)
    qa, qb = _edge4(y2, src1, dst1, zeros8)
    return _tc3_call(qa, qb, invd, r2, batch2d, g2, b2, bl2r)[0]


# speedup vs baseline: 19.8984x; 1.0087x over previous
import jax
import jax.numpy as jnp
from jax import lax
from jax.experimental import pallas as pl
from jax.experimental.pallas import tpu as pltpu
from jax.experimental.pallas import tpu_sc as plsc

N = 49152
E = 786432
NB = 12

_NROWS_TILE = N // 16


def _gather_scatter_pipeline(tbl, src, dst, acc, base, nchunks, ch,
                             s0, s1, d0, d1, r0, r1, gs0, gs1, ss0, ss1):
    pltpu.sync_copy(src.at[pl.ds(base, ch)], s0)
    pltpu.sync_copy(dst.at[pl.ds(base, ch)], d0)
    pltpu.async_copy(tbl.at[s0], r0, gs0)

    def body(j, carry):
        i0 = 2 * j
        @pl.when(j > 0)
        def _():
            pltpu.make_async_copy(r1, acc.at[d1], ss1).wait()
        pltpu.sync_copy(src.at[pl.ds(base + (i0 + 1) * ch, ch)], s1)
        pltpu.sync_copy(dst.at[pl.ds(base + (i0 + 1) * ch, ch)], d1)
        pltpu.async_copy(tbl.at[s1], r1, gs1)
        pltpu.make_async_copy(tbl.at[s0], r0, gs0).wait()
        pltpu.async_copy(r0, acc.at[d0], ss0, add=True)

        @pl.when(j < nchunks // 2 - 1)
        def _():
            pltpu.make_async_copy(r0, acc.at[d0], ss0).wait()
            pltpu.sync_copy(src.at[pl.ds(base + (i0 + 2) * ch, ch)], s0)
            pltpu.sync_copy(dst.at[pl.ds(base + (i0 + 2) * ch, ch)], d0)
            pltpu.async_copy(tbl.at[s0], r0, gs0)
        pltpu.make_async_copy(tbl.at[s1], r1, gs1).wait()
        pltpu.async_copy(r1, acc.at[d1], ss1, add=True)
        return carry

    lax.fori_loop(0, nchunks // 2, body, 0)
    pltpu.make_async_copy(r0, acc.at[d0], ss0).wait()
    pltpu.make_async_copy(r1, acc.at[d1], ss1).wait()


def _zero_acc(zeros, acc, s, bounce, wb):
    def z(j, carry):
        t0 = s * _NROWS_TILE + j * wb
        pltpu.sync_copy(zeros.at[pl.ds(t0, wb)], bounce)
        pltpu.sync_copy(bounce, acc.at[pl.ds(t0, wb)])
        return carry
    lax.fori_loop(0, _NROWS_TILE // wb, z, 0)


_E_T4 = E // 32
_CH4 = 2048
_WB4 = 1536


def _edge4_body(table, src, dst, zeros, out_a, out_b,
                s0, s1, d0, d1, r0, r1, acc, gs0, gs1, ss0, ss1):
    c = lax.axis_index("c")
    s = lax.axis_index("s")
    bounce = r0.at[pl.ds(0, _WB4)]
    _zero_acc(zeros, acc, s, bounce, _WB4)
    plsc.subcore_barrier()

    base = (c * 16 + s) * _E_T4
    _gather_scatter_pipeline(table, src, dst, acc, base,
                             _E_T4 // _CH4, _CH4,
                             s0, s1, d0, d1, r0, r1, gs0, gs1, ss0, ss1)
    plsc.subcore_barrier()

    def wb(j, carry):
        t0 = s * _NROWS_TILE + j * _WB4
        pltpu.sync_copy(acc.at[pl.ds(t0, _WB4)], bounce)

        @pl.when(c == 0)
        def _():
            pltpu.sync_copy(bounce, out_a.at[pl.ds(t0, _WB4)])

        @pl.when(c == 1)
        def _():
            pltpu.sync_copy(bounce, out_b.at[pl.ds(t0, _WB4)])
        return carry

    lax.fori_loop(0, _NROWS_TILE // _WB4, wb, 0)


_edge4 = pl.kernel(
    _edge4_body,
    out_type=[jax.ShapeDtypeStruct((N, 8), jnp.float32)] * 2,
    mesh=plsc.VectorSubcoreMesh(core_axis_name="c", subcore_axis_name="s",
                                num_cores=2, num_subcores=16),
    compiler_params=pltpu.CompilerParams(use_tc_tiling_on_sc=False),
    scratch_types=[
        pltpu.VMEM((_CH4,), jnp.int32),
        pltpu.VMEM((_CH4,), jnp.int32),
        pltpu.VMEM((_CH4,), jnp.int32),
        pltpu.VMEM((_CH4,), jnp.int32),
        pltpu.VMEM((_CH4, 8), jnp.float32),
        pltpu.VMEM((_CH4, 8), jnp.float32),
        pltpu.VMEM_SHARED((N, 8), jnp.float32),
        pltpu.SemaphoreType.DMA,
        pltpu.SemaphoreType.DMA,
        pltpu.SemaphoreType.DMA,
        pltpu.SemaphoreType.DMA,
    ],
)


_E_T32 = E // 16
_CH32 = 384
_WB32 = 384


def _edge32_body(h_a, h_b, src, dst, zeros, out_a, out_b,
                 s0, s1, d0, d1, r0, r1, acc, gs0, gs1, ss0, ss1):
    c = lax.axis_index("c")
    s = lax.axis_index("s")
    _zero_acc(zeros, acc, s, r0, _WB32)
    plsc.subcore_barrier()

    base = s * _E_T32
    nchunks = _E_T32 // _CH32

    @pl.when(c == 0)
    def _():
        _gather_scatter_pipeline(h_a, src, dst, acc, base, nchunks, _CH32,
                                 s0, s1, d0, d1, r0, r1, gs0, gs1, ss0, ss1)

    @pl.when(c == 1)
    def _():
        _gather_scatter_pipeline(h_b, src, dst, acc, base, nchunks, _CH32,
                                 s0, s1, d0, d1, r0, r1, gs0, gs1, ss0, ss1)

    plsc.subcore_barrier()

    def wb(j, carry):
        t0 = s * _NROWS_TILE + j * _WB32
        pltpu.sync_copy(acc.at[pl.ds(t0, _WB32)], r0)

        @pl.when(c == 0)
        def _():
            pltpu.sync_copy(r0, out_a.at[pl.ds(t0, _WB32)])

        @pl.when(c == 1)
        def _():
            pltpu.sync_copy(r0, out_b.at[pl.ds(t0, _WB32)])
        return carry

    lax.fori_loop(0, _NROWS_TILE // _WB32, wb, 0)


_edge32 = pl.kernel(
    _edge32_body,
    out_type=[jax.ShapeDtypeStruct((N, 32), jnp.float32)] * 2,
    mesh=plsc.VectorSubcoreMesh(core_axis_name="c", subcore_axis_name="s",
                                num_cores=2, num_subcores=16),
    compiler_params=pltpu.CompilerParams(use_tc_tiling_on_sc=False),
    scratch_types=[
        pltpu.VMEM((_CH32,), jnp.int32),
        pltpu.VMEM((_CH32,), jnp.int32),
        pltpu.VMEM((_CH32,), jnp.int32),
        pltpu.VMEM((_CH32,), jnp.int32),
        pltpu.VMEM((_CH32, 32), jnp.float32),
        pltpu.VMEM((_CH32, 32), jnp.float32),
        pltpu.VMEM_SHARED((N, 32), jnp.float32),
        pltpu.SemaphoreType.DMA,
        pltpu.SemaphoreType.DMA,
        pltpu.SemaphoreType.DMA,
        pltpu.SemaphoreType.DMA,
    ],
)


_BN = 4096


def _silu(x):
    return x * jax.nn.sigmoid(x)


def _tmod_body(t_ref, w1_ref, b1_ref, w2_ref, b2_ref, o_ref):
    h = t_ref[...] * w1_ref[...] + b1_ref[...]
    h = _silu(h)
    o_ref[...] = (
        jnp.dot(h, w2_ref[...], preferred_element_type=jnp.float32)
        + b2_ref[...]
    )


def _onehot(bt):
    return (bt == lax.broadcasted_iota(jnp.int32, (_BN, NB), 1)).astype(
        jnp.float32
    )


def _tc1_body(x_ref, pa_ref, pb_ref, bt_ref, g_ref, be_ref,
              wl_ref, wr_ref, bl_ref, ha_ref, hb_ref, invd_ref):
    p = pa_ref[...] + pb_ref[...]
    invd = 1.0 / jnp.maximum(p[:, 3:4], 1.0)
    agg = p[:, 0:3] * invd
    h = (jnp.dot(agg, wl_ref[...], preferred_element_type=jnp.float32)
         + jnp.dot(x_ref[...], wr_ref[...], preferred_element_type=jnp.float32)
         + bl_ref[...])
    oh = _onehot(bt_ref[...])
    gm = jnp.dot(oh, g_ref[...], preferred_element_type=jnp.float32)
    bb = jnp.dot(oh, be_ref[...], preferred_element_type=jnp.float32)
    h = _silu(h * (1.0 + gm) + bb)
    ha_ref[...] = h[:, :32]
    hb_ref[...] = h[:, 32:]
    invd_ref[...] = invd


def _tc2_body(aa_ref, ab_ref, ha_ref, hb_ref, invd_ref, bt_ref,
              g_ref, be_ref, wl_ref, wr_ref, bl_ref, wl2_ref, wr2_ref,
              y_ref, r_ref):
    agg = jnp.concatenate([aa_ref[...], ab_ref[...]], axis=1) * invd_ref[...]
    h1 = jnp.concatenate([ha_ref[...], hb_ref[...]], axis=1)
    h = (jnp.dot(agg, wl_ref[...], preferred_element_type=jnp.float32)
         + jnp.dot(h1, wr_ref[...], preferred_element_type=jnp.float32)
         + bl_ref[...])
    oh = _onehot(bt_ref[...])
    gm = jnp.dot(oh, g_ref[...], preferred_element_type=jnp.float32)
    bb = jnp.dot(oh, be_ref[...], preferred_element_type=jnp.float32)
    h = _silu(h * (1.0 + gm) + bb)
    y_ref[...] = jnp.dot(h, wl2_ref[...], preferred_element_type=jnp.float32)
    r_ref[...] = jnp.dot(h, wr2_ref[...], preferred_element_type=jnp.float32)


def _tc3_body(qa_ref, qb_ref, invd_ref, r_ref, bt_ref,
              g_ref, be_ref, bl_ref, o_ref):
    agg = (qa_ref[...] + qb_ref[...])[:, 0:3] * invd_ref[...]
    pre = agg + bl_ref[...] + r_ref[...][:, 0:3]
    oh = _onehot(bt_ref[...])
    gm = jnp.dot(oh, g_ref[...], preferred_element_type=jnp.float32)
    bb = jnp.dot(oh, be_ref[...], preferred_element_type=jnp.float32)
    o_ref[...] = pre * (1.0 + gm) + bb


def _col(shape):
    return pl.BlockSpec((_BN,) + shape[1:],
                        lambda i: (i,) + (0,) * (len(shape) - 1))


def _full(shape):
    return pl.BlockSpec(shape, lambda i: (0,) * len(shape))


def _tc_call(body, in_shapes, out_shapes):
    return pl.pallas_call(
        body,
        grid=(N // _BN,),
        in_specs=[_col(s) if s[0] == N else _full(s) for s in in_shapes],
        out_specs=[_col(s) for s in out_shapes],
        out_shape=[jax.ShapeDtypeStruct(s, jnp.float32) for s in out_shapes],
    )


_tmod_call = pl.pallas_call(
    _tmod_body,
    out_shape=jax.ShapeDtypeStruct((NB, 262), jnp.float32),
)

_tc1_call = _tc_call(
    _tc1_body,
    [(N, 3), (N, 8), (N, 8), (N, 1), (NB, 64), (NB, 64), (3, 64), (3, 64),
     (1, 64)],
    [(N, 32), (N, 32), (N, 1)],
)

_tc2_call = _tc_call(
    _tc2_body,
    [(N, 32), (N, 32), (N, 32), (N, 32), (N, 1), (N, 1), (NB, 64), (NB, 64),
     (64, 64), (64, 64), (1, 64), (64, 8), (64, 4)],
    [(N, 8), (N, 4)],
)

_tc3_call = _tc_call(
    _tc3_body,
    [(N, 8), (N, 8), (N, 1), (N, 4), (N, 1), (NB, 3), (NB, 3), (1, 3)],
    [(N, 3)],
)


def kernel(x, edge_index, batch, t, Wl0, bl0, Wr0, Wl1, bl1, Wr1,
           Wl2, bl2, Wr2, Wt1, bt1, Wt2, bt2):
    f32 = jnp.float32
    src1, dst1 = edge_index[0], edge_index[1]
    batch2d = batch.reshape(N, 1)
    x8 = jnp.concatenate([x, jnp.ones((N, 1), f32),
                          jnp.zeros((N, 4), f32)], axis=1)
    zeros8 = jnp.zeros((N, 8), f32)
    zeros32 = jnp.zeros((N, 32), f32)
    t2 = t.reshape(NB, 1)

    wl0t, wr0t = Wl0.T, Wr0.T
    wl1t, wr1t = Wl1.T, Wr1.T
    wl2t8 = jnp.pad(Wl2.T, ((0, 0), (0, 5)))
    wr2t4 = jnp.pad(Wr2.T, ((0, 0), (0, 1)))
    bl0r, bl1r, bl2r = bl0.reshape(1, 64), bl1.reshape(1, 64), bl2.reshape(1, 3)
    wt1t, wt2t = Wt1.T, Wt2.T
    bt1r, bt2r = bt1.reshape(1, 128), bt2.reshape(1, 262)

    tmod = _tmod_call(t2, wt1t, bt1r, wt2t, bt2r)
    g0, b0 = tmod[:, 0:64], tmod[:, 64:128]
    g1, b1 = tmod[:, 128:192], tmod[:, 192:256]
    g2, b2 = tmod[:, 256:259], tmod[:, 259:262]

    pa, pb = _edge4(x8, src1, dst1, zeros8)
    ha, hb, invd = _tc1_call(x, pa, pb, batch2d, g0, b0, wl0t, wr0t, bl0r)
    aa, ab = _edge32(ha, hb, src1, dst1, zeros32)
    y2, r2 = _tc2_call(aa, ab, ha, hb, invd, batch2d, g1, b1,
                       wl1t, wr1t, bl1r, wl2t8, wr2t4)
    qa, qb = _edge4(y2, src1, dst1, zeros8)
    return _tc3_call(qa, qb, invd, r2, batch2d, g2, b2, bl2r)[0]


# speedup vs baseline: 22.7631x; 1.1440x over previous
import jax
import jax.numpy as jnp
from jax import lax
from jax.experimental import pallas as pl
from jax.experimental.pallas import tpu as pltpu
from jax.experimental.pallas import tpu_sc as plsc

N = 49152
E = 786432
NB = 12

_NROWS_TILE = N // 16


def _gather_scatter_pipeline(tbl, src, dst, acc, base, nchunks, ch,
                             sb, db, rr, gs, ss, si, di):
    nj = nchunks // 4

    def ld(i, b):
        pltpu.async_copy(src.at[pl.ds(base + i * ch, ch)], sb[b], si[b])
        pltpu.async_copy(dst.at[pl.ds(base + i * ch, ch)], db[b], di[b])

    ld(0, 0)
    ld(1, 1)
    pltpu.make_async_copy(src.at[pl.ds(base, ch)], sb[0], si[0]).wait()
    pltpu.async_copy(tbl.at[sb[0]], rr[0], gs[0])

    def body(j, carry):
        for k in range(4):
            kp, kq = k % 2, (k + 1) % 2
            if k == 0:
                @pl.when(j > 0)
                def _():
                    pltpu.make_async_copy(rr[1], acc.at[db[3]], ss[1]).wait()
            else:
                pltpu.make_async_copy(rr[kq], acc.at[db[k - 1]], ss[kq]).wait()
            if k < 2:
                ld(4 * j + k + 2, k + 2)
            else:
                @pl.when(j < nj - 1)
                def _():
                    ld(4 * j + k + 2, (k + 2) % 4)
            if k < 3:
                pltpu.make_async_copy(
                    src.at[pl.ds(base, ch)], sb[k + 1], si[k + 1]).wait()
                pltpu.async_copy(tbl.at[sb[k + 1]], rr[kq], gs[kq])
            else:
                @pl.when(j < nj - 1)
                def _():
                    pltpu.make_async_copy(
                        src.at[pl.ds(base, ch)], sb[0], si[0]).wait()
                    pltpu.async_copy(tbl.at[sb[0]], rr[kq], gs[kq])
            pltpu.make_async_copy(tbl.at[sb[k]], rr[kp], gs[kp]).wait()
            pltpu.make_async_copy(
                dst.at[pl.ds(base, ch)], db[k], di[k]).wait()
            pltpu.async_copy(rr[kp], acc.at[db[k]], ss[kp], add=True)
        return carry

    lax.fori_loop(0, nj, body, 0)
    pltpu.make_async_copy(rr[1], acc.at[db[3]], ss[1]).wait()


def _zero_acc(zeros, acc, s, bounce, wb):
    def z(j, carry):
        t0 = s * _NROWS_TILE + j * wb
        pltpu.sync_copy(zeros.at[pl.ds(t0, wb)], bounce)
        pltpu.sync_copy(bounce, acc.at[pl.ds(t0, wb)])
        return carry
    lax.fori_loop(0, _NROWS_TILE // wb, z, 0)


_E_T4 = E // 32
_CH4 = 2048
_WB4 = 1536


def _edge4_body(table, src, dst, zeros, out_a, out_b,
                sb0, sb1, sb2, sb3, db0, db1, db2, db3, r0, r1, acc,
                gs0, gs1, ss0, ss1, si0, si1, si2, si3, di0, di1, di2, di3):
    c = lax.axis_index("c")
    s = lax.axis_index("s")
    bounce = r0.at[pl.ds(0, _WB4)]
    _zero_acc(zeros, acc, s, bounce, _WB4)
    plsc.subcore_barrier()

    base = (c * 16 + s) * _E_T4
    _gather_scatter_pipeline(table, src, dst, acc, base, _E_T4 // _CH4, _CH4,
                             [sb0, sb1, sb2, sb3], [db0, db1, db2, db3],
                             [r0, r1], [gs0, gs1], [ss0, ss1],
                             [si0, si1, si2, si3], [di0, di1, di2, di3])
    plsc.subcore_barrier()

    def wb(j, carry):
        t0 = s * _NROWS_TILE + j * _WB4
        pltpu.sync_copy(acc.at[pl.ds(t0, _WB4)], bounce)

        @pl.when(c == 0)
        def _():
            pltpu.sync_copy(bounce, out_a.at[pl.ds(t0, _WB4)])

        @pl.when(c == 1)
        def _():
            pltpu.sync_copy(bounce, out_b.at[pl.ds(t0, _WB4)])
        return carry

    lax.fori_loop(0, _NROWS_TILE // _WB4, wb, 0)


def _sc_scratch(ch, w):
    return ([pltpu.VMEM((ch,), jnp.int32)] * 8
            + [pltpu.VMEM((ch, w), jnp.float32)] * 2
            + [pltpu.VMEM_SHARED((N, w), jnp.float32)]
            + [pltpu.SemaphoreType.DMA] * 12)


_edge4 = pl.kernel(
    _edge4_body,
    out_type=[jax.ShapeDtypeStruct((N, 8), jnp.float32)] * 2,
    mesh=plsc.VectorSubcoreMesh(core_axis_name="c", subcore_axis_name="s",
                                num_cores=2, num_subcores=16),
    compiler_params=pltpu.CompilerParams(use_tc_tiling_on_sc=False),
    scratch_types=_sc_scratch(_CH4, 8),
)


_E_T32 = E // 16
_CH32 = 384
_WB32 = 384


def _edge32_body(h_a, h_b, src, dst, zeros, out_a, out_b,
                 sb0, sb1, sb2, sb3, db0, db1, db2, db3, r0, r1, acc,
                 gs0, gs1, ss0, ss1, si0, si1, si2, si3, di0, di1, di2, di3):
    c = lax.axis_index("c")
    s = lax.axis_index("s")
    _zero_acc(zeros, acc, s, r0, _WB32)
    plsc.subcore_barrier()

    base = s * _E_T32
    nchunks = _E_T32 // _CH32
    args = ([sb0, sb1, sb2, sb3], [db0, db1, db2, db3], [r0, r1],
            [gs0, gs1], [ss0, ss1], [si0, si1, si2, si3],
            [di0, di1, di2, di3])

    @pl.when(c == 0)
    def _():
        _gather_scatter_pipeline(h_a, src, dst, acc, base, nchunks, _CH32,
                                 *args)

    @pl.when(c == 1)
    def _():
        _gather_scatter_pipeline(h_b, src, dst, acc, base, nchunks, _CH32,
                                 *args)

    plsc.subcore_barrier()

    def wb(j, carry):
        t0 = s * _NROWS_TILE + j * _WB32
        pltpu.sync_copy(acc.at[pl.ds(t0, _WB32)], r0)

        @pl.when(c == 0)
        def _():
            pltpu.sync_copy(r0, out_a.at[pl.ds(t0, _WB32)])

        @pl.when(c == 1)
        def _():
            pltpu.sync_copy(r0, out_b.at[pl.ds(t0, _WB32)])
        return carry

    lax.fori_loop(0, _NROWS_TILE // _WB32, wb, 0)


_edge32 = pl.kernel(
    _edge32_body,
    out_type=[jax.ShapeDtypeStruct((N, 32), jnp.float32)] * 2,
    mesh=plsc.VectorSubcoreMesh(core_axis_name="c", subcore_axis_name="s",
                                num_cores=2, num_subcores=16),
    compiler_params=pltpu.CompilerParams(use_tc_tiling_on_sc=False),
    scratch_types=_sc_scratch(_CH32, 32),
)


_BN = 4096


def _silu(x):
    return x * jax.nn.sigmoid(x)


def _tmod_body(t_ref, w1_ref, b1_ref, w2_ref, b2_ref, o_ref):
    h = t_ref[...] * w1_ref[...] + b1_ref[...]
    h = _silu(h)
    o_ref[...] = (
        jnp.dot(h, w2_ref[...], preferred_element_type=jnp.float32)
        + b2_ref[...]
    )


def _onehot(bt):
    return (bt == lax.broadcasted_iota(jnp.int32, (_BN, NB), 1)).astype(
        jnp.float32
    )


def _tc1_body(x_ref, pa_ref, pb_ref, bt_ref, g_ref, be_ref,
              wl_ref, wr_ref, bl_ref, ha_ref, hb_ref, invd_ref):
    p = pa_ref[...] + pb_ref[...]
    invd = 1.0 / jnp.maximum(p[:, 3:4], 1.0)
    agg = p[:, 0:3] * invd
    h = (jnp.dot(agg, wl_ref[...], preferred_element_type=jnp.float32)
         + jnp.dot(x_ref[...], wr_ref[...], preferred_element_type=jnp.float32)
         + bl_ref[...])
    oh = _onehot(bt_ref[...])
    gm = jnp.dot(oh, g_ref[...], preferred_element_type=jnp.float32)
    bb = jnp.dot(oh, be_ref[...], preferred_element_type=jnp.float32)
    h = _silu(h * (1.0 + gm) + bb)
    ha_ref[...] = h[:, :32]
    hb_ref[...] = h[:, 32:]
    invd_ref[...] = invd


def _tc2_body(aa_ref, ab_ref, ha_ref, hb_ref, invd_ref, bt_ref,
              g_ref, be_ref, wl_ref, wr_ref, bl_ref, wl2_ref, wr2_ref,
              y_ref, r_ref):
    agg = jnp.concatenate([aa_ref[...], ab_ref[...]], axis=1) * invd_ref[...]
    h1 = jnp.concatenate([ha_ref[...], hb_ref[...]], axis=1)
    h = (jnp.dot(agg, wl_ref[...], preferred_element_type=jnp.float32)
         + jnp.dot(h1, wr_ref[...], preferred_element_type=jnp.float32)
         + bl_ref[...])
    oh = _onehot(bt_ref[...])
    gm = jnp.dot(oh, g_ref[...], preferred_element_type=jnp.float32)
    bb = jnp.dot(oh, be_ref[...], preferred_element_type=jnp.float32)
    h = _silu(h * (1.0 + gm) + bb)
    y_ref[...] = jnp.dot(h, wl2_ref[...], preferred_element_type=jnp.float32)
    r_ref[...] = jnp.dot(h, wr2_ref[...], preferred_element_type=jnp.float32)


def _tc3_body(qa_ref, qb_ref, invd_ref, r_ref, bt_ref,
              g_ref, be_ref, bl_ref, o_ref):
    agg = (qa_ref[...] + qb_ref[...])[:, 0:3] * invd_ref[...]
    pre = agg + bl_ref[...] + r_ref[...][:, 0:3]
    oh = _onehot(bt_ref[...])
    gm = jnp.dot(oh, g_ref[...], preferred_element_type=jnp.float32)
    bb = jnp.dot(oh, be_ref[...], preferred_element_type=jnp.float32)
    o_ref[...] = pre * (1.0 + gm) + bb


def _col(shape):
    return pl.BlockSpec((_BN,) + shape[1:],
                        lambda i: (i,) + (0,) * (len(shape) - 1))


def _full(shape):
    return pl.BlockSpec(shape, lambda i: (0,) * len(shape))


def _tc_call(body, in_shapes, out_shapes):
    return pl.pallas_call(
        body,
        grid=(N // _BN,),
        in_specs=[_col(s) if s[0] == N else _full(s) for s in in_shapes],
        out_specs=[_col(s) for s in out_shapes],
        out_shape=[jax.ShapeDtypeStruct(s, jnp.float32) for s in out_shapes],
    )


_tmod_call = pl.pallas_call(
    _tmod_body,
    out_shape=jax.ShapeDtypeStruct((NB, 262), jnp.float32),
)

_tc1_call = _tc_call(
    _tc1_body,
    [(N, 3), (N, 8), (N, 8), (N, 1), (NB, 64), (NB, 64), (3, 64), (3, 64),
     (1, 64)],
    [(N, 32), (N, 32), (N, 1)],
)

_tc2_call = _tc_call(
    _tc2_body,
    [(N, 32), (N, 32), (N, 32), (N, 32), (N, 1), (N, 1), (NB, 64), (NB, 64),
     (64, 64), (64, 64), (1, 64), (64, 8), (64, 4)],
    [(N, 8), (N, 4)],
)

_tc3_call = _tc_call(
    _tc3_body,
    [(N, 8), (N, 8), (N, 1), (N, 4), (N, 1), (NB, 3), (NB, 3), (1, 3)],
    [(N, 3)],
)


def kernel(x, edge_index, batch, t, Wl0, bl0, Wr0, Wl1, bl1, Wr1,
           Wl2, bl2, Wr2, Wt1, bt1, Wt2, bt2):
    f32 = jnp.float32
    src1, dst1 = edge_index[0], edge_index[1]
    batch2d = batch.reshape(N, 1)
    x8 = jnp.concatenate([x, jnp.ones((N, 1), f32),
                          jnp.zeros((N, 4), f32)], axis=1)
    zeros8 = jnp.zeros((N, 8), f32)
    zeros32 = jnp.zeros((N, 32), f32)
    t2 = t.reshape(NB, 1)

    wl0t, wr0t = Wl0.T, Wr0.T
    wl1t, wr1t = Wl1.T, Wr1.T
    wl2t8 = jnp.pad(Wl2.T, ((0, 0), (0, 5)))
    wr2t4 = jnp.pad(Wr2.T, ((0, 0), (0, 1)))
    bl0r, bl1r, bl2r = bl0.reshape(1, 64), bl1.reshape(1, 64), bl2.reshape(1, 3)
    wt1t, wt2t = Wt1.T, Wt2.T
    bt1r, bt2r = bt1.reshape(1, 128), bt2.reshape(1, 262)

    tmod = _tmod_call(t2, wt1t, bt1r, wt2t, bt2r)
    g0, b0 = tmod[:, 0:64], tmod[:, 64:128]
    g1, b1 = tmod[:, 128:192], tmod[:, 192:256]
    g2, b2 = tmod[:, 256:259], tmod[:, 259:262]

    pa, pb = _edge4(x8, src1, dst1, zeros8)
    ha, hb, invd = _tc1_call(x, pa, pb, batch2d, g0, b0, wl0t, wr0t, bl0r)
    aa, ab = _edge32(ha, hb, src1, dst1, zeros32)
    y2, r2 = _tc2_call(aa, ab, ha, hb, invd, batch2d, g1, b1,
                       wl1t, wr1t, bl1r, wl2t8, wr2t4)
    qa, qb = _edge4(y2, src1, dst1, zeros8)
    return _tc3_call(qa, qb, invd, r2, batch2d, g2, b2, bl2r)[0]


# speedup vs baseline: 23.5004x; 1.0324x over previous
import jax
import jax.numpy as jnp
from jax import lax
from jax.experimental import pallas as pl
from jax.experimental.pallas import tpu as pltpu
from jax.experimental.pallas import tpu_sc as plsc

N = 49152
E = 786432
NB = 12

_NROWS_TILE = N // 16


def _gather_scatter_pipeline(tbl, src, dst, acc, base, nchunks, ch,
                             sb, db, rr, gs, ss, si, di):
    nj = nchunks // 4

    def ld(i, b):
        pltpu.async_copy(src.at[pl.ds(base + i * ch, ch)], sb[b], si[b])
        pltpu.async_copy(dst.at[pl.ds(base + i * ch, ch)], db[b], di[b])

    def prologue():
        ld(0, 0)
        ld(1, 1)
        pltpu.make_async_copy(src.at[pl.ds(base, ch)], sb[0], si[0]).wait()
        pltpu.async_copy(tbl.at[sb[0]], rr[0], gs[0])

    def body(j, carry):
        for k in range(4):
            kp, kq = k % 2, (k + 1) % 2
            if k == 0:
                @pl.when(j > 0)
                def _():
                    pltpu.make_async_copy(rr[1], acc.at[db[3]], ss[1]).wait()
            else:
                pltpu.make_async_copy(rr[kq], acc.at[db[k - 1]], ss[kq]).wait()
            if k < 2:
                ld(4 * j + k + 2, k + 2)
            else:
                @pl.when(j < nj - 1)
                def _():
                    ld(4 * j + k + 2, (k + 2) % 4)
            if k < 3:
                pltpu.make_async_copy(
                    src.at[pl.ds(base, ch)], sb[k + 1], si[k + 1]).wait()
                pltpu.async_copy(tbl.at[sb[k + 1]], rr[kq], gs[kq])
            else:
                @pl.when(j < nj - 1)
                def _():
                    pltpu.make_async_copy(
                        src.at[pl.ds(base, ch)], sb[0], si[0]).wait()
                    pltpu.async_copy(tbl.at[sb[0]], rr[kq], gs[kq])
            pltpu.make_async_copy(tbl.at[sb[k]], rr[kp], gs[kp]).wait()
            pltpu.make_async_copy(
                dst.at[pl.ds(base, ch)], db[k], di[k]).wait()
            pltpu.async_copy(rr[kp], acc.at[db[k]], ss[kp], add=True)
        return carry

    def run():
        lax.fori_loop(0, nj, body, 0)
        pltpu.make_async_copy(rr[1], acc.at[db[3]], ss[1]).wait()

    return prologue, run


def _zero_acc(zeros, acc, s, bounce, wb):
    def z(j, carry):
        t0 = s * _NROWS_TILE + j * wb
        pltpu.sync_copy(zeros.at[pl.ds(t0, wb)], bounce)
        pltpu.sync_copy(bounce, acc.at[pl.ds(t0, wb)])
        return carry
    lax.fori_loop(0, _NROWS_TILE // wb, z, 0)


_E_T4 = E // 32
_CH4 = 3072
_WB4 = 3072


def _edge4_body(table, src, dst, zeros, out_a, out_b,
                sb0, sb1, sb2, sb3, db0, db1, db2, db3, r0, r1, acc,
                gs0, gs1, ss0, ss1, si0, si1, si2, si3, di0, di1, di2, di3):
    c = lax.axis_index("c")
    s = lax.axis_index("s")
    base = (c * 16 + s) * _E_T4
    prologue, run = _gather_scatter_pipeline(
        table, src, dst, acc, base, _E_T4 // _CH4, _CH4,
        [sb0, sb1, sb2, sb3], [db0, db1, db2, db3],
        [r0, r1], [gs0, gs1], [ss0, ss1],
        [si0, si1, si2, si3], [di0, di1, di2, di3])
    prologue()
    t0 = s * _NROWS_TILE
    pltpu.sync_copy(zeros.at[pl.ds(t0, _NROWS_TILE)], r1)
    pltpu.sync_copy(r1, acc.at[pl.ds(t0, _NROWS_TILE)])
    plsc.subcore_barrier()
    run()
    plsc.subcore_barrier()

    pltpu.sync_copy(acc.at[pl.ds(t0, _NROWS_TILE)], r1)

    @pl.when(c == 0)
    def _():
        pltpu.sync_copy(r1, out_a.at[pl.ds(t0, _NROWS_TILE)])

    @pl.when(c == 1)
    def _():
        pltpu.sync_copy(r1, out_b.at[pl.ds(t0, _NROWS_TILE)])


def _sc_scratch(ch, w):
    return ([pltpu.VMEM((ch,), jnp.int32)] * 8
            + [pltpu.VMEM((ch, w), jnp.float32)] * 2
            + [pltpu.VMEM_SHARED((N, w), jnp.float32)]
            + [pltpu.SemaphoreType.DMA] * 12)


_edge4 = pl.kernel(
    _edge4_body,
    out_type=[jax.ShapeDtypeStruct((N, 8), jnp.float32)] * 2,
    mesh=plsc.VectorSubcoreMesh(core_axis_name="c", subcore_axis_name="s",
                                num_cores=2, num_subcores=16),
    compiler_params=pltpu.CompilerParams(use_tc_tiling_on_sc=False),
    scratch_types=_sc_scratch(_CH4, 8),
)


_E_T32 = E // 16
_CH32 = 384
_WB32 = 384


def _edge32_body(h_a, h_b, src, dst, out_a, out_b,
                 sb0, sb1, sb2, sb3, db0, db1, db2, db3, r0, r1, acc,
                 gs0, gs1, ss0, ss1, si0, si1, si2, si3, di0, di1, di2, di3):
    c = lax.axis_index("c")
    s = lax.axis_index("s")
    base = s * _E_T32
    nchunks = _E_T32 // _CH32
    args = ([sb0, sb1, sb2, sb3], [db0, db1, db2, db3], [r0, r1],
            [gs0, gs1], [ss0, ss1], [si0, si1, si2, si3],
            [di0, di1, di2, di3])

    zv = jnp.zeros((16,), jnp.float32)

    def vz(i, carry):
        r1[i, pl.ds(0, 16)] = zv
        r1[i, pl.ds(16, 16)] = zv
        return carry

    lax.fori_loop(0, _CH32, vz, 0)
    nwb = _NROWS_TILE // _WB32
    for j in range(nwb):
        t0 = s * _NROWS_TILE + j * _WB32
        pltpu.async_copy(r1, acc.at[pl.ds(t0, _WB32)], gs0 if j % 2 == 0
                         else gs1)
    for j in range(nwb):
        pltpu.make_async_copy(
            r1, acc.at[pl.ds(0, _WB32)], gs0 if j % 2 == 0 else gs1).wait()

    def run_for(tbl):
        prologue, run = _gather_scatter_pipeline(
            tbl, src, dst, acc, base, nchunks, _CH32, *args)
        prologue()
        plsc.subcore_barrier()
        run()

    @pl.when(c == 0)
    def _():
        run_for(h_a)

    @pl.when(c == 1)
    def _():
        run_for(h_b)

    plsc.subcore_barrier()

    for j in range(_NROWS_TILE // _WB32):
        t0 = s * _NROWS_TILE + j * _WB32
        bank = r0 if j % 2 == 0 else r1
        sem = gs0 if j % 2 == 0 else gs1
        if j >= 2:
            pltpu.make_async_copy(bank, out_a.at[pl.ds(0, _WB32)], sem).wait()
        pltpu.sync_copy(acc.at[pl.ds(t0, _WB32)], bank)

        @pl.when(c == 0)
        def _(bank=bank, sem=sem, t0=t0):
            pltpu.async_copy(bank, out_a.at[pl.ds(t0, _WB32)], sem)

        @pl.when(c == 1)
        def _(bank=bank, sem=sem, t0=t0):
            pltpu.async_copy(bank, out_b.at[pl.ds(t0, _WB32)], sem)

    pltpu.make_async_copy(r0, out_a.at[pl.ds(0, _WB32)], gs0).wait()
    pltpu.make_async_copy(r1, out_a.at[pl.ds(0, _WB32)], gs1).wait()


_edge32 = pl.kernel(
    _edge32_body,
    out_type=[jax.ShapeDtypeStruct((N, 32), jnp.float32)] * 2,
    mesh=plsc.VectorSubcoreMesh(core_axis_name="c", subcore_axis_name="s",
                                num_cores=2, num_subcores=16),
    compiler_params=pltpu.CompilerParams(use_tc_tiling_on_sc=False),
    scratch_types=_sc_scratch(_CH32, 32),
)


_BN = 4096


def _silu(x):
    return x * jax.nn.sigmoid(x)


def _tmod_body(t_ref, w1_ref, b1_ref, w2_ref, b2_ref, o_ref):
    h = t_ref[...] * w1_ref[...] + b1_ref[...]
    h = _silu(h)
    o_ref[...] = (
        jnp.dot(h, w2_ref[...], preferred_element_type=jnp.float32)
        + b2_ref[...]
    )


def _onehot(bt):
    return (bt == lax.broadcasted_iota(jnp.int32, (_BN, NB), 1)).astype(
        jnp.float32
    )


def _tc1_body(x_ref, pa_ref, pb_ref, bt_ref, g_ref, be_ref,
              wl_ref, wr_ref, bl_ref, ha_ref, hb_ref, invd_ref):
    p = pa_ref[...] + pb_ref[...]
    invd = 1.0 / jnp.maximum(p[:, 3:4], 1.0)
    agg = p[:, 0:3] * invd
    h = (jnp.dot(agg, wl_ref[...], preferred_element_type=jnp.float32)
         + jnp.dot(x_ref[...], wr_ref[...], preferred_element_type=jnp.float32)
         + bl_ref[...])
    oh = _onehot(bt_ref[...])
    gm = jnp.dot(oh, g_ref[...], preferred_element_type=jnp.float32)
    bb = jnp.dot(oh, be_ref[...], preferred_element_type=jnp.float32)
    h = _silu(h * (1.0 + gm) + bb)
    ha_ref[...] = h[:, :32]
    hb_ref[...] = h[:, 32:]
    invd_ref[...] = invd


def _tc2_body(aa_ref, ab_ref, ha_ref, hb_ref, invd_ref, bt_ref,
              g_ref, be_ref, wl_ref, wr_ref, bl_ref, wl2_ref, wr2_ref,
              y_ref, r_ref):
    agg = jnp.concatenate([aa_ref[...], ab_ref[...]], axis=1) * invd_ref[...]
    h1 = jnp.concatenate([ha_ref[...], hb_ref[...]], axis=1)
    h = (jnp.dot(agg, wl_ref[...], preferred_element_type=jnp.float32)
         + jnp.dot(h1, wr_ref[...], preferred_element_type=jnp.float32)
         + bl_ref[...])
    oh = _onehot(bt_ref[...])
    gm = jnp.dot(oh, g_ref[...], preferred_element_type=jnp.float32)
    bb = jnp.dot(oh, be_ref[...], preferred_element_type=jnp.float32)
    h = _silu(h * (1.0 + gm) + bb)
    y_ref[...] = jnp.dot(h, wl2_ref[...], preferred_element_type=jnp.float32)
    r_ref[...] = jnp.dot(h, wr2_ref[...], preferred_element_type=jnp.float32)


def _tc3_body(qa_ref, qb_ref, invd_ref, r_ref, bt_ref,
              g_ref, be_ref, bl_ref, o_ref):
    agg = (qa_ref[...] + qb_ref[...])[:, 0:3] * invd_ref[...]
    pre = agg + bl_ref[...] + r_ref[...][:, 0:3]
    oh = _onehot(bt_ref[...])
    gm = jnp.dot(oh, g_ref[...], preferred_element_type=jnp.float32)
    bb = jnp.dot(oh, be_ref[...], preferred_element_type=jnp.float32)
    o_ref[...] = pre * (1.0 + gm) + bb


def _col(shape):
    return pl.BlockSpec((_BN,) + shape[1:],
                        lambda i: (i,) + (0,) * (len(shape) - 1))


def _full(shape):
    return pl.BlockSpec(shape, lambda i: (0,) * len(shape))


def _tc_call(body, in_shapes, out_shapes):
    return pl.pallas_call(
        body,
        grid=(N // _BN,),
        in_specs=[_col(s) if s[0] == N else _full(s) for s in in_shapes],
        out_specs=[_col(s) for s in out_shapes],
        out_shape=[jax.ShapeDtypeStruct(s, jnp.float32) for s in out_shapes],
    )


_tmod_call = pl.pallas_call(
    _tmod_body,
    out_shape=jax.ShapeDtypeStruct((NB, 262), jnp.float32),
)

_tc1_call = _tc_call(
    _tc1_body,
    [(N, 3), (N, 8), (N, 8), (N, 1), (NB, 64), (NB, 64), (3, 64), (3, 64),
     (1, 64)],
    [(N, 32), (N, 32), (N, 1)],
)

_tc2_call = _tc_call(
    _tc2_body,
    [(N, 32), (N, 32), (N, 32), (N, 32), (N, 1), (N, 1), (NB, 64), (NB, 64),
     (64, 64), (64, 64), (1, 64), (64, 8), (64, 4)],
    [(N, 8), (N, 4)],
)

_tc3_call = _tc_call(
    _tc3_body,
    [(N, 8), (N, 8), (N, 1), (N, 4), (N, 1), (NB, 3), (NB, 3), (1, 3)],
    [(N, 3)],
)


def kernel(x, edge_index, batch, t, Wl0, bl0, Wr0, Wl1, bl1, Wr1,
           Wl2, bl2, Wr2, Wt1, bt1, Wt2, bt2):
    f32 = jnp.float32
    src1, dst1 = edge_index[0], edge_index[1]
    batch2d = batch.reshape(N, 1)
    x8 = jnp.concatenate([x, jnp.ones((N, 1), f32),
                          jnp.zeros((N, 4), f32)], axis=1)
    zeros8 = jnp.zeros((N, 8), f32)
    t2 = t.reshape(NB, 1)

    wl0t, wr0t = Wl0.T, Wr0.T
    wl1t, wr1t = Wl1.T, Wr1.T
    wl2t8 = jnp.pad(Wl2.T, ((0, 0), (0, 5)))
    wr2t4 = jnp.pad(Wr2.T, ((0, 0), (0, 1)))
    bl0r, bl1r, bl2r = bl0.reshape(1, 64), bl1.reshape(1, 64), bl2.reshape(1, 3)
    wt1t, wt2t = Wt1.T, Wt2.T
    bt1r, bt2r = bt1.reshape(1, 128), bt2.reshape(1, 262)

    tmod = _tmod_call(t2, wt1t, bt1r, wt2t, bt2r)
    g0, b0 = tmod[:, 0:64], tmod[:, 64:128]
    g1, b1 = tmod[:, 128:192], tmod[:, 192:256]
    g2, b2 = tmod[:, 256:259], tmod[:, 259:262]

    pa, pb = _edge4(x8, src1, dst1, zeros8)
    ha, hb, invd = _tc1_call(x, pa, pb, batch2d, g0, b0, wl0t, wr0t, bl0r)
    aa, ab = _edge32(ha, hb, src1, dst1)
    y2, r2 = _tc2_call(aa, ab, ha, hb, invd, batch2d, g1, b1,
                       wl1t, wr1t, bl1r, wl2t8, wr2t4)
    qa, qb = _edge4(y2, src1, dst1, zeros8)
    return _tc3_call(qa, qb, invd, r2, batch2d, g2, b2, bl2r)[0]


# speedup vs baseline: 28.8921x; 1.2294x over previous
import jax
import jax.numpy as jnp
from jax import lax
from jax.experimental import pallas as pl
from jax.experimental.pallas import tpu as pltpu
from jax.experimental.pallas import tpu_sc as plsc

N = 49152
E = 786432
NB = 12

_NROWS_TILE = N // 16


def _gather_scatter_pipeline(tbl, src, dst, acc, base, nchunks, ch,
                             sb, db, rr, gs, ss, si, di):
    nj = nchunks // 4

    def ld(i, b):
        pltpu.async_copy(src.at[pl.ds(base + i * ch, ch)], sb[b], si[b])
        pltpu.async_copy(dst.at[pl.ds(base + i * ch, ch)], db[b], di[b])

    def prologue():
        ld(0, 0)
        ld(1, 1)
        pltpu.make_async_copy(src.at[pl.ds(base, ch)], sb[0], si[0]).wait()
        pltpu.async_copy(tbl.at[sb[0]], rr[0], gs[0])

    def body(j, carry):
        for k in range(4):
            kp, kq = k % 2, (k + 1) % 2
            if k == 0:
                @pl.when(j > 0)
                def _():
                    pltpu.make_async_copy(rr[1], acc.at[db[3]], ss[1]).wait()
            else:
                pltpu.make_async_copy(rr[kq], acc.at[db[k - 1]], ss[kq]).wait()
            if k < 2:
                ld(4 * j + k + 2, k + 2)
            else:
                @pl.when(j < nj - 1)
                def _():
                    ld(4 * j + k + 2, (k + 2) % 4)
            if k < 3:
                pltpu.make_async_copy(
                    src.at[pl.ds(base, ch)], sb[k + 1], si[k + 1]).wait()
                pltpu.async_copy(tbl.at[sb[k + 1]], rr[kq], gs[kq])
            else:
                @pl.when(j < nj - 1)
                def _():
                    pltpu.make_async_copy(
                        src.at[pl.ds(base, ch)], sb[0], si[0]).wait()
                    pltpu.async_copy(tbl.at[sb[0]], rr[kq], gs[kq])
            pltpu.make_async_copy(tbl.at[sb[k]], rr[kp], gs[kp]).wait()
            pltpu.make_async_copy(
                dst.at[pl.ds(base, ch)], db[k], di[k]).wait()
            pltpu.async_copy(rr[kp], acc.at[db[k]], ss[kp], add=True)
        return carry

    def run():
        lax.fori_loop(0, nj, body, 0)
        pltpu.make_async_copy(rr[1], acc.at[db[3]], ss[1]).wait()

    return prologue, run


def _zero_acc(zeros, acc, s, bounce, wb):
    def z(j, carry):
        t0 = s * _NROWS_TILE + j * wb
        pltpu.sync_copy(zeros.at[pl.ds(t0, wb)], bounce)
        pltpu.sync_copy(bounce, acc.at[pl.ds(t0, wb)])
        return carry
    lax.fori_loop(0, _NROWS_TILE // wb, z, 0)


_E_T4 = E // 32
_CH4 = 3072
_WB4 = 3072


def _edge4_body(table, src, dst, zeros, out,
                sb0, sb1, sb2, sb3, db0, db1, db2, db3, r0, r1, acc,
                gs0, gs1, ss0, ss1, si0, si1, si2, si3, di0, di1, di2, di3):
    c = lax.axis_index("c")
    s = lax.axis_index("s")
    base = (c * 16 + s) * _E_T4
    prologue, run = _gather_scatter_pipeline(
        table, src, dst, acc, base, _E_T4 // _CH4, _CH4,
        [sb0, sb1, sb2, sb3], [db0, db1, db2, db3],
        [r0, r1], [gs0, gs1], [ss0, ss1],
        [si0, si1, si2, si3], [di0, di1, di2, di3])
    prologue()
    t0 = s * _NROWS_TILE
    pltpu.sync_copy(zeros.at[pl.ds(t0, _NROWS_TILE)], r1)
    pltpu.sync_copy(r1, acc.at[pl.ds(t0, _NROWS_TILE)])
    plsc.subcore_barrier()
    run()
    plsc.subcore_barrier()

    pltpu.sync_copy(acc.at[pl.ds(t0, _NROWS_TILE)], r1)

    @pl.when(c == 0)
    def _():
        pltpu.sync_copy(r1, out.at[pl.ds(t0, _NROWS_TILE), pl.ds(0, 8)])

    @pl.when(c == 1)
    def _():
        pltpu.sync_copy(r1, out.at[pl.ds(t0, _NROWS_TILE), pl.ds(8, 8)])


def _sc_scratch(ch, w):
    return ([pltpu.VMEM((ch,), jnp.int32)] * 8
            + [pltpu.VMEM((ch, w), jnp.float32)] * 2
            + [pltpu.VMEM_SHARED((N, w), jnp.float32)]
            + [pltpu.SemaphoreType.DMA] * 12)


_edge4 = pl.kernel(
    _edge4_body,
    out_type=[jax.ShapeDtypeStruct((N, 128), jnp.float32)],
    mesh=plsc.VectorSubcoreMesh(core_axis_name="c", subcore_axis_name="s",
                                num_cores=2, num_subcores=16),
    compiler_params=pltpu.CompilerParams(use_tc_tiling_on_sc=False),
    scratch_types=_sc_scratch(_CH4, 8),
)


_E_T32 = E // 16
_CH32 = 384
_WB32 = 384


def _edge32_body(h_a, h_b, src, dst, out,
                 sb0, sb1, sb2, sb3, db0, db1, db2, db3, r0, r1, acc,
                 gs0, gs1, ss0, ss1, si0, si1, si2, si3, di0, di1, di2, di3):
    c = lax.axis_index("c")
    s = lax.axis_index("s")
    base = s * _E_T32
    nchunks = _E_T32 // _CH32
    args = ([sb0, sb1, sb2, sb3], [db0, db1, db2, db3], [r0, r1],
            [gs0, gs1], [ss0, ss1], [si0, si1, si2, si3],
            [di0, di1, di2, di3])

    zv = jnp.zeros((16,), jnp.float32)

    def vz(i, carry):
        r1[i, pl.ds(0, 16)] = zv
        r1[i, pl.ds(16, 16)] = zv
        return carry

    lax.fori_loop(0, _CH32, vz, 0)
    nwb = _NROWS_TILE // _WB32
    for j in range(nwb):
        t0 = s * _NROWS_TILE + j * _WB32
        pltpu.async_copy(r1, acc.at[pl.ds(t0, _WB32)], gs0 if j % 2 == 0
                         else gs1)
    for j in range(nwb):
        pltpu.make_async_copy(
            r1, acc.at[pl.ds(0, _WB32)], gs0 if j % 2 == 0 else gs1).wait()

    def run_for(tbl):
        prologue, run = _gather_scatter_pipeline(
            tbl, src, dst, acc, base, nchunks, _CH32, *args)
        prologue()
        plsc.subcore_barrier()
        run()

    @pl.when(c == 0)
    def _():
        run_for(h_a)

    @pl.when(c == 1)
    def _():
        run_for(h_b)

    plsc.subcore_barrier()

    for j in range(_NROWS_TILE // _WB32):
        t0 = s * _NROWS_TILE + j * _WB32
        bank = r0 if j % 2 == 0 else r1
        sem = gs0 if j % 2 == 0 else gs1
        if j >= 2:
            pltpu.make_async_copy(
                bank, out.at[pl.ds(0, _WB32), pl.ds(0, 32)], sem).wait()
        pltpu.sync_copy(acc.at[pl.ds(t0, _WB32)], bank)

        @pl.when(c == 0)
        def _(bank=bank, sem=sem, t0=t0):
            pltpu.async_copy(bank, out.at[pl.ds(t0, _WB32), pl.ds(0, 32)],
                             sem)

        @pl.when(c == 1)
        def _(bank=bank, sem=sem, t0=t0):
            pltpu.async_copy(bank, out.at[pl.ds(t0, _WB32), pl.ds(32, 32)],
                             sem)

    pltpu.make_async_copy(r0, out.at[pl.ds(0, _WB32), pl.ds(0, 32)],
                          gs0).wait()
    pltpu.make_async_copy(r1, out.at[pl.ds(0, _WB32), pl.ds(0, 32)],
                          gs1).wait()


_edge32 = pl.kernel(
    _edge32_body,
    out_type=[jax.ShapeDtypeStruct((N, 128), jnp.float32)],
    mesh=plsc.VectorSubcoreMesh(core_axis_name="c", subcore_axis_name="s",
                                num_cores=2, num_subcores=16),
    compiler_params=pltpu.CompilerParams(use_tc_tiling_on_sc=False),
    scratch_types=_sc_scratch(_CH32, 32),
)


_BN = 4096


def _silu(x):
    return x * jax.nn.sigmoid(x)


def _tmod_body(t_ref, w1_ref, b1_ref, w2_ref, b2_ref, o_ref):
    h = t_ref[...] * w1_ref[...] + b1_ref[...]
    h = _silu(h)
    o_ref[...] = (
        jnp.dot(h, w2_ref[...], preferred_element_type=jnp.float32)
        + b2_ref[...]
    )


def _onehot(bt):
    return (bt == lax.broadcasted_iota(jnp.int32, (_BN, NB), 1)).astype(
        jnp.float32
    )


def _tc1_body(x_ref, pw_ref, bt_ref, g_ref, be_ref,
              wl_ref, wr_ref, bl_ref, ha_ref, hb_ref, invd_ref):
    pw = pw_ref[...]
    p = pw[:, 0:8] + pw[:, 8:16]
    invd = 1.0 / jnp.maximum(p[:, 3:4], 1.0)
    agg = p[:, 0:3] * invd
    h = (jnp.dot(agg, wl_ref[...], preferred_element_type=jnp.float32)
         + jnp.dot(x_ref[...], wr_ref[...], preferred_element_type=jnp.float32)
         + bl_ref[...])
    oh = _onehot(bt_ref[...])
    gm = jnp.dot(oh, g_ref[...], preferred_element_type=jnp.float32)
    bb = jnp.dot(oh, be_ref[...], preferred_element_type=jnp.float32)
    h = _silu(h * (1.0 + gm) + bb)
    ha_ref[...] = h[:, :32]
    hb_ref[...] = h[:, 32:]
    invd_ref[...] = invd


def _tc2_body(aw_ref, ha_ref, hb_ref, invd_ref, bt_ref,
              g_ref, be_ref, wl_ref, wr_ref, bl_ref, wl2_ref, wr2_ref,
              y_ref, r_ref):
    agg = aw_ref[...][:, 0:64] * invd_ref[...]
    h1 = jnp.concatenate([ha_ref[...], hb_ref[...]], axis=1)
    h = (jnp.dot(agg, wl_ref[...], preferred_element_type=jnp.float32)
         + jnp.dot(h1, wr_ref[...], preferred_element_type=jnp.float32)
         + bl_ref[...])
    oh = _onehot(bt_ref[...])
    gm = jnp.dot(oh, g_ref[...], preferred_element_type=jnp.float32)
    bb = jnp.dot(oh, be_ref[...], preferred_element_type=jnp.float32)
    h = _silu(h * (1.0 + gm) + bb)
    y_ref[...] = jnp.dot(h, wl2_ref[...], preferred_element_type=jnp.float32)
    r_ref[...] = jnp.dot(h, wr2_ref[...], preferred_element_type=jnp.float32)


def _tc3_body(qw_ref, invd_ref, r_ref, bt_ref,
              g_ref, be_ref, bl_ref, o_ref):
    qw = qw_ref[...]
    agg = (qw[:, 0:8] + qw[:, 8:16])[:, 0:3] * invd_ref[...]
    pre = agg + bl_ref[...] + r_ref[...][:, 0:3]
    oh = _onehot(bt_ref[...])
    gm = jnp.dot(oh, g_ref[...], preferred_element_type=jnp.float32)
    bb = jnp.dot(oh, be_ref[...], preferred_element_type=jnp.float32)
    o_ref[...] = pre * (1.0 + gm) + bb


def _col(shape):
    return pl.BlockSpec((_BN,) + shape[1:],
                        lambda i: (i,) + (0,) * (len(shape) - 1))


def _full(shape):
    return pl.BlockSpec(shape, lambda i: (0,) * len(shape))


def _tc_call(body, in_shapes, out_shapes):
    return pl.pallas_call(
        body,
        grid=(N // _BN,),
        in_specs=[_col(s) if s[0] == N else _full(s) for s in in_shapes],
        out_specs=[_col(s) for s in out_shapes],
        out_shape=[jax.ShapeDtypeStruct(s, jnp.float32) for s in out_shapes],
    )


_tmod_call = pl.pallas_call(
    _tmod_body,
    out_shape=jax.ShapeDtypeStruct((NB, 262), jnp.float32),
)

_tc1_call = _tc_call(
    _tc1_body,
    [(N, 3), (N, 128), (N, 1), (NB, 64), (NB, 64), (3, 64), (3, 64),
     (1, 64)],
    [(N, 32), (N, 32), (N, 1)],
)

_tc2_call = _tc_call(
    _tc2_body,
    [(N, 128), (N, 32), (N, 32), (N, 1), (N, 1), (NB, 64), (NB, 64),
     (64, 64), (64, 64), (1, 64), (64, 8), (64, 4)],
    [(N, 8), (N, 4)],
)

_tc3_call = _tc_call(
    _tc3_body,
    [(N, 128), (N, 1), (N, 4), (N, 1), (NB, 3), (NB, 3), (1, 3)],
    [(N, 3)],
)


def kernel(x, edge_index, batch, t, Wl0, bl0, Wr0, Wl1, bl1, Wr1,
           Wl2, bl2, Wr2, Wt1, bt1, Wt2, bt2):
    f32 = jnp.float32
    src1, dst1 = edge_index[0], edge_index[1]
    batch2d = batch.reshape(N, 1)
    x8 = jnp.concatenate([x, jnp.ones((N, 1), f32),
                          jnp.zeros((N, 4), f32)], axis=1)
    zeros8 = jnp.zeros((N, 8), f32)
    t2 = t.reshape(NB, 1)

    wl0t, wr0t = Wl0.T, Wr0.T
    wl1t, wr1t = Wl1.T, Wr1.T
    wl2t8 = jnp.pad(Wl2.T, ((0, 0), (0, 5)))
    wr2t4 = jnp.pad(Wr2.T, ((0, 0), (0, 1)))
    bl0r, bl1r, bl2r = bl0.reshape(1, 64), bl1.reshape(1, 64), bl2.reshape(1, 3)
    wt1t, wt2t = Wt1.T, Wt2.T
    bt1r, bt2r = bt1.reshape(1, 128), bt2.reshape(1, 262)

    tmod = _tmod_call(t2, wt1t, bt1r, wt2t, bt2r)
    g0, b0 = tmod[:, 0:64], tmod[:, 64:128]
    g1, b1 = tmod[:, 128:192], tmod[:, 192:256]
    g2, b2 = tmod[:, 256:259], tmod[:, 259:262]

    pw = _edge4(x8, src1, dst1, zeros8)[0]
    ha, hb, invd = _tc1_call(x, pw, batch2d, g0, b0, wl0t, wr0t, bl0r)
    aw = _edge32(ha, hb, src1, dst1)[0]
    y2, r2 = _tc2_call(aw, ha, hb, invd, batch2d, g1, b1,
                       wl1t, wr1t, bl1r, wl2t8, wr2t4)
    qw = _edge4(y2, src1, dst1, zeros8)[0]
    return _tc3_call(qw, invd, r2, batch2d, g2, b2, bl2r)[0]
